# Initial kernel scaffold; baseline (speedup 1.0000x reference)
import functools
import math

import jax
import jax.numpy as jnp
from jax import lax
from jax.experimental import pallas as pl
from jax.experimental.pallas import tpu as pltpu
from jax.experimental.pallas import tpu_sc as plsc

_LOG_NORM_CONST = -0.5 * math.log(2 * math.pi)
_NUM_ITER = 3
_BETA = 0.9
_K = 512
_D = 64
_N = 2048

_HIGH = jax.lax.Precision.HIGHEST


def _dot(a, b):
    return jax.lax.dot_general(
        a, b, (((1,), (0,)), ((), ())),
        precision=_HIGH, preferred_element_type=jnp.float32)


def _dot_t(a, b):
    return jax.lax.dot_general(
        a, b, (((0,), (0,)), ((), ())),
        precision=_HIGH, preferred_element_type=jnp.float32)


def _ll_terms(mu, logvar):
    iv = jnp.exp(-logvar)
    a = mu * iv
    c = (_D * _LOG_NORM_CONST
         - 0.5 * jnp.sum(logvar, axis=1)
         - 0.5 * jnp.sum(mu * a, axis=1))
    return iv, a, c


def _tc_body(xf_ref, bmu_ref, blv_ref, emb_ref, enc_ref, embnew_ref, idx_ref):
    xf = xf_ref[:]
    xsq = xf * xf
    mu = bmu_ref[:]
    logvar = blv_ref[:]

    for _ in range(_NUM_ITER):
        iv, a, c = _ll_terms(mu, logvar)
        ll = (_dot(xf, a.T) - 0.5 * _dot(xsq, iv.T)) + c[None, :]
        m = jnp.max(ll, axis=1, keepdims=True)
        s = jnp.sum(jnp.exp(ll - m), axis=1, keepdims=True)
        lse = m + jnp.log(s)
        p = jnp.exp(ll - lse)
        n_k = jnp.sum(p, axis=0)[:, None]
        s1 = _dot_t(p, xf)
        s2 = _dot_t(p, xsq)
        denom = n_k + 1e-6
        mu_new = s1 / denom
        var = (s2 - 2.0 * mu_new * s1 + n_k * (mu_new * mu_new)) / denom
        logvar = jnp.log(jnp.maximum(var, 1e-6))
        mu = mu_new

    iv, a, c = _ll_terms(mu, logvar)
    ll = (_dot(xf, a.T) - 0.5 * _dot(xsq, iv.T)) + c[None, :]
    lik = jnp.exp(ll)
    mx = jnp.max(lik, axis=1, keepdims=True)
    kio = lax.broadcasted_iota(jnp.int32, (_N, _K), 1)
    idx = jnp.min(jnp.where(lik == mx, kio, _K), axis=1)
    enc_ref[:] = (kio == idx[:, None]).astype(jnp.float32)
    idx_ref[:] = idx[:, None]
    embnew_ref[:] = _BETA * emb_ref[:] + (1.0 - _BETA) * mu


def _tc_em(xf, batch_mu, batch_logvar, emb_mu):
    return pl.pallas_call(
        _tc_body,
        out_shape=[
            jax.ShapeDtypeStruct((_N, _K), jnp.float32),
            jax.ShapeDtypeStruct((_K, _D), jnp.float32),
            jax.ShapeDtypeStruct((_N, 1), jnp.int32),
        ],
    )(xf, batch_mu, batch_logvar, emb_mu)


_NW = 32
_BPW = _N // _NW


def _sc_lookup(table, idx):
    nc = 2
    mesh = plsc.VectorSubcoreMesh(core_axis_name="c", subcore_axis_name="s")

    @functools.partial(
        pl.kernel,
        mesh=mesh,
        out_type=jax.ShapeDtypeStruct((_N, _D), jnp.float32),
        scratch_types=[
            pltpu.VMEM((_BPW,), jnp.int32),
            pltpu.VMEM((_BPW, _D), jnp.float32),
            pltpu.SemaphoreType.DMA,
        ],
    )
    def gather_k(table_hbm, idx_hbm, out_hbm, idx_v, rows_v, sem):
        wid = lax.axis_index("s") * nc + lax.axis_index("c")
        base = wid * _BPW
        pltpu.sync_copy(idx_hbm.at[pl.ds(base, _BPW)], idx_v)
        pltpu.async_copy(table_hbm.at[idx_v], rows_v, sem).wait()
        pltpu.sync_copy(rows_v, out_hbm.at[pl.ds(base, _BPW)])

    return gather_k(table, idx)


def kernel(x, embeddings_mu, embeddings_logvar, embeddings_pi, batch_mu,
           batch_logvar):
    del embeddings_logvar, embeddings_pi
    b, ch, h, w = x.shape
    xf = jnp.transpose(x, (0, 2, 3, 1)).reshape(-1, _D)
    enc, emb_new, idx = _tc_em(xf, batch_mu, batch_logvar, embeddings_mu)
    quantized = _sc_lookup(emb_new, idx.reshape(-1))
    qr = jnp.transpose(quantized.reshape(b, h, w, ch), (0, 3, 1, 2))
    return enc, qr


# speedup vs baseline: 7.6270x; 7.6270x over previous
import functools
import math

import jax
import jax.numpy as jnp
from jax import lax
from jax.experimental import pallas as pl
from jax.experimental.pallas import tpu as pltpu
from jax.experimental.pallas import tpu_sc as plsc

_LOG_NORM_CONST = -0.5 * math.log(2 * math.pi)
_NUM_ITER = 3
_BETA = 0.9
_K = 512
_D = 64
_N = 2048

_HIGH = jax.lax.Precision.HIGHEST


def _dot(a, b):
    return jax.lax.dot_general(
        a, b, (((1,), (0,)), ((), ())),
        precision=_HIGH, preferred_element_type=jnp.float32)


def _dot_t(a, b):
    return jax.lax.dot_general(
        a, b, (((0,), (0,)), ((), ())),
        precision=_HIGH, preferred_element_type=jnp.float32)


def _ll_terms(mu, logvar):
    iv = jnp.exp(-logvar)
    a = mu * iv
    c = (_D * _LOG_NORM_CONST
         - 0.5 * jnp.sum(logvar, axis=1)
         - 0.5 * jnp.sum(mu * a, axis=1))
    return iv, a, c


def _tc_body(xf_ref, bmu_ref, blv_ref, emb_ref, enc_ref, embnew_ref, idx_ref):
    xf = xf_ref[:]
    xsq = xf * xf
    mu = bmu_ref[:]
    logvar = blv_ref[:]

    for _ in range(_NUM_ITER):
        iv, a, c = _ll_terms(mu, logvar)
        ll = (_dot(xf, a.T) - 0.5 * _dot(xsq, iv.T)) + c[None, :]
        m = jnp.max(ll, axis=1, keepdims=True)
        s = jnp.sum(jnp.exp(ll - m), axis=1, keepdims=True)
        lse = m + jnp.log(s)
        p = jnp.exp(ll - lse)
        n_k = jnp.sum(p, axis=0)[:, None]
        s1 = _dot_t(p, xf)
        s2 = _dot_t(p, xsq)
        denom = n_k + 1e-6
        mu_new = s1 / denom
        var = (s2 - 2.0 * mu_new * s1 + n_k * (mu_new * mu_new)) / denom
        logvar = jnp.log(jnp.maximum(var, 1e-6))
        mu = mu_new

    iv, a, c = _ll_terms(mu, logvar)
    ll = (_dot(xf, a.T) - 0.5 * _dot(xsq, iv.T)) + c[None, :]
    lik = jnp.exp(ll)
    mx = jnp.max(lik, axis=1, keepdims=True)
    kio = lax.broadcasted_iota(jnp.int32, (_N, _K), 1)
    idx = jnp.min(jnp.where(lik == mx, kio, _K), axis=1)
    enc_ref[:] = (kio == idx[:, None]).astype(jnp.float32)
    idx_ref[:] = idx[:, None]
    embnew_ref[:] = _BETA * emb_ref[:] + (1.0 - _BETA) * mu


def _tc_em(xf, batch_mu, batch_logvar, emb_mu):
    return pl.pallas_call(
        _tc_body,
        out_shape=[
            jax.ShapeDtypeStruct((_N, _K), jnp.float32),
            jax.ShapeDtypeStruct((_K, _D), jnp.float32),
            jax.ShapeDtypeStruct((_N, 1), jnp.int32),
        ],
    )(xf, batch_mu, batch_logvar, emb_mu)


_NW = 32
_BPW = _N // _NW


_DP = 128


def _sc_lookup(table, idx):
    nc = 2
    mesh = plsc.VectorSubcoreMesh(core_axis_name="c", subcore_axis_name="s")

    @functools.partial(
        pl.kernel,
        mesh=mesh,
        out_type=jax.ShapeDtypeStruct((_N, _DP), jnp.float32),
        scratch_types=[
            pltpu.VMEM((_BPW,), jnp.int32),
            pltpu.VMEM((_BPW, _DP), jnp.float32),
            pltpu.SemaphoreType.DMA,
        ],
    )
    def gather_k(table_hbm, idx_hbm, out_hbm, idx_v, rows_v, sem):
        wid = lax.axis_index("s") * nc + lax.axis_index("c")
        base = wid * _BPW
        pltpu.sync_copy(idx_hbm.at[pl.ds(base, _BPW)], idx_v)
        pltpu.async_copy(table_hbm.at[idx_v], rows_v, sem).wait()
        pltpu.sync_copy(rows_v, out_hbm.at[pl.ds(base, _BPW)])

    padded = jnp.pad(table, ((0, 0), (0, _DP - _D)))
    return gather_k(padded, idx)[:, :_D]


def kernel(x, embeddings_mu, embeddings_logvar, embeddings_pi, batch_mu,
           batch_logvar):
    del embeddings_logvar, embeddings_pi
    b, ch, h, w = x.shape
    xf = jnp.transpose(x, (0, 2, 3, 1)).reshape(-1, _D)
    enc, emb_new, idx = _tc_em(xf, batch_mu, batch_logvar, embeddings_mu)
    quantized = _sc_lookup(emb_new, idx.reshape(-1))
    qr = jnp.transpose(quantized.reshape(b, h, w, ch), (0, 3, 1, 2))
    return enc, qr
